# Initial kernel scaffold; baseline (speedup 1.0000x reference)
import jax
import jax.numpy as jnp
from jax.experimental import pallas as pl


def kernel(x, edge_index, W1, b1, W2, b2):
    raise NotImplementedError("write your pallas kernel here")


# speedup vs baseline: 27.3497x; 27.3497x over previous
import functools

import jax
import jax.numpy as jnp
from jax import lax
from jax.experimental import pallas as pl
from jax.experimental.pallas import tpu as pltpu
from jax.experimental.pallas import tpu_sc as plsc

N = 50000
E = 1_600_000
F_IN = 3703
F_H = 16
F_OUT = 6
F_OUT_P = 8

NC, NS = 2, 16
NW = NC * NS

ROW_T = 512
GRID_N = 98
P = 50432

CH = 128
NCHUNK = 392
EPT = NCHUNK * CH
KG = 8
NGROUP = NCHUNK // KG
EP = EPT * NW

_MESH = plsc.VectorSubcoreMesh(
    core_axis_name="c", subcore_axis_name="s", num_cores=NC, num_subcores=NS)


@functools.partial(
    pl.kernel,
    out_type=jax.ShapeDtypeStruct((NW, P), jnp.float32),
    mesh=_MESH,
    scratch_types=[
        pltpu.VMEM((EPT,), jnp.int32),
        pltpu.VMEM((P,), jnp.float32),
    ],
    compiler_params=pltpu.CompilerParams(needs_layout_passes=False),
)
def _deg_kernel(dst_hbm, deg_hbm, idx_v, acc_v):
    wid = lax.axis_index("s") * NC + lax.axis_index("c")
    zero16 = jnp.zeros((16,), jnp.float32)
    ones16 = jnp.ones((16,), jnp.float32)

    def zbody(i, carry):
        acc_v[pl.ds(i * 16, 16)] = zero16
        return carry
    lax.fori_loop(0, P // 16, zbody, 0)

    pltpu.sync_copy(dst_hbm.at[wid], idx_v)

    def sbody(i, carry):
        idx16 = idx_v[pl.ds(i * 16, 16)]
        plsc.addupdate_scatter(acc_v, [idx16], ones16)
        return carry
    lax.fori_loop(0, EPT // 16, sbody, 0)

    pltpu.sync_copy(acc_v, deg_hbm.at[wid])


def _make_agg(width):
    @functools.partial(
        pl.kernel,
        out_type=jax.ShapeDtypeStruct((NC, P, width), jnp.float32),
        mesh=_MESH,
        scratch_types=[
            pltpu.VMEM((KG, CH), jnp.int32),
            pltpu.VMEM((KG, CH), jnp.int32),
            pltpu.VMEM((KG, CH, width), jnp.float32),
            pltpu.VMEM_SHARED((P, width), jnp.float32),
            pltpu.SemaphoreType.DMA,
        ],
        compiler_params=pltpu.CompilerParams(use_tc_tiling_on_sc=False),
    )
    def _agg(feat_hbm, src_hbm, dst_hbm, zeros_hbm, out_hbm,
             sidx_v, didx_v, rows_v, acc_s, gsem):
        cid = lax.axis_index("c")
        sid = lax.axis_index("s")
        wid = sid * NC + cid
        rows_per = P // NS
        roff = pl.multiple_of(sid * rows_per, 8)

        pltpu.sync_copy(zeros_hbm.at[pl.ds(roff, rows_per)],
                        acc_s.at[pl.ds(roff, rows_per)])
        plsc.subcore_barrier()

        def group(g, carry):
            base = g * KG
            pltpu.sync_copy(src_hbm.at[wid, pl.ds(base, KG)], sidx_v)
            pltpu.sync_copy(dst_hbm.at[wid, pl.ds(base, KG)], didx_v)
            descs = []
            for j in range(KG):
                descs.append(pltpu.async_copy(
                    feat_hbm.at[sidx_v.at[j]], rows_v.at[j], gsem))
            for j in range(KG):
                descs[j].wait()
            for j in range(KG):
                pltpu.sync_copy(rows_v.at[j], acc_s.at[didx_v.at[j]],
                                add=True)
            return carry
        lax.fori_loop(0, NGROUP, group, 0)

        plsc.subcore_barrier()
        pltpu.sync_copy(acc_s.at[pl.ds(roff, rows_per)],
                        out_hbm.at[cid, pl.ds(roff, rows_per)])
    return _agg


_agg16 = _make_agg(F_H)
_agg8 = _make_agg(F_OUT_P)


def _mm_body(x_ref, w_ref, degp_ref, hp_ref, dinv_ref):
    deg = jnp.sum(degp_ref[...], axis=0) + 1.0
    dinv = lax.rsqrt(deg)
    h = jnp.dot(x_ref[...], w_ref[...], preferred_element_type=jnp.float32)
    hp_ref[...] = h * dinv[:, None]
    dinv_ref[...] = dinv[:, None]


def _mm_prep(x, W1, deg_parts):
    return pl.pallas_call(
        _mm_body,
        grid=(GRID_N,),
        in_specs=[
            pl.BlockSpec((ROW_T, F_IN), lambda i: (i, 0)),
            pl.BlockSpec((F_IN, F_H), lambda i: (0, 0)),
            pl.BlockSpec((NW, ROW_T), lambda i: (0, i)),
        ],
        out_specs=[
            pl.BlockSpec((ROW_T, F_H), lambda i: (i, 0)),
            pl.BlockSpec((ROW_T, 1), lambda i: (i, 0)),
        ],
        out_shape=[
            jax.ShapeDtypeStruct((N, F_H), jnp.float32),
            jax.ShapeDtypeStruct((N, 1), jnp.float32),
        ],
    )(x, W1, deg_parts)


def _mid_body(agg_ref, hp_ref, dinv_ref, w2_ref, b1_ref, gp_ref):
    s = agg_ref[0] + agg_ref[1] + hp_ref[...]
    dinv = dinv_ref[...]
    out1 = s * dinv + b1_ref[...]
    r = jnp.maximum(out1, 0.0)
    g = jnp.dot(r, w2_ref[...], preferred_element_type=jnp.float32)
    gp_ref[...] = g * dinv


def _mid(agg1, hp, dinv, W2p, b1r):
    return pl.pallas_call(
        _mid_body,
        grid=(GRID_N,),
        in_specs=[
            pl.BlockSpec((NC, ROW_T, F_H), lambda i: (0, i, 0)),
            pl.BlockSpec((ROW_T, F_H), lambda i: (i, 0)),
            pl.BlockSpec((ROW_T, 1), lambda i: (i, 0)),
            pl.BlockSpec((F_H, F_OUT_P), lambda i: (0, 0)),
            pl.BlockSpec((1, F_H), lambda i: (0, 0)),
        ],
        out_specs=pl.BlockSpec((ROW_T, F_OUT_P), lambda i: (i, 0)),
        out_shape=jax.ShapeDtypeStruct((N, F_OUT_P), jnp.float32),
    )(agg1, hp, dinv, W2p, b1r)


def _final_body(agg_ref, gp_ref, dinv_ref, b2_ref, out_ref):
    s = agg_ref[0] + agg_ref[1] + gp_ref[...]
    z = s * dinv_ref[...] + b2_ref[...]
    col = lax.broadcasted_iota(jnp.int32, (ROW_T, F_OUT_P), 1)
    zm = jnp.where(col < F_OUT, z, -1e30)
    m = jnp.max(zm, axis=1, keepdims=True)
    ez = jnp.exp(zm - m)
    ssum = jnp.sum(ez, axis=1, keepdims=True)
    ls = (z - m) - jnp.log(ssum)
    out_ref[...] = ls[:, :F_OUT]


def _final(agg2, gp, dinv, b2r):
    return pl.pallas_call(
        _final_body,
        grid=(GRID_N,),
        in_specs=[
            pl.BlockSpec((NC, ROW_T, F_OUT_P), lambda i: (0, i, 0)),
            pl.BlockSpec((ROW_T, F_OUT_P), lambda i: (i, 0)),
            pl.BlockSpec((ROW_T, 1), lambda i: (i, 0)),
            pl.BlockSpec((1, F_OUT_P), lambda i: (0, 0)),
        ],
        out_specs=pl.BlockSpec((ROW_T, F_OUT), lambda i: (i, 0)),
        out_shape=jax.ShapeDtypeStruct((N, F_OUT), jnp.float32),
    )(agg2, gp, dinv, b2r)


def kernel(x, edge_index, W1, b1, W2, b2):
    fill = jnp.full((EP - E,), N, jnp.int32)
    srcp = jnp.concatenate([edge_index[0], fill]).reshape(NW, NCHUNK, CH)
    dstp = jnp.concatenate([edge_index[1], fill]).reshape(NW, NCHUNK, CH)
    dstf = dstp.reshape(NW, EPT)

    deg_parts = _deg_kernel(dstf)

    hp, dinv = _mm_prep(x, W1, deg_parts)
    hp_p = jnp.concatenate(
        [hp, jnp.zeros((P - N, F_H), jnp.float32)], axis=0)

    z16 = jnp.zeros((P, F_H), jnp.float32)
    agg1 = _agg16(hp_p, srcp, dstp, z16)

    W2p = jnp.zeros((F_H, F_OUT_P), jnp.float32).at[:, :F_OUT].set(W2)
    b1r = b1.reshape(1, F_H)
    gp = _mid(agg1, hp, dinv, W2p, b1r)
    gp_p = jnp.concatenate(
        [gp, jnp.zeros((P - N, F_OUT_P), jnp.float32)], axis=0)

    z8 = jnp.zeros((P, F_OUT_P), jnp.float32)
    agg2 = _agg8(gp_p, srcp, dstp, z8)

    b2r = jnp.zeros((1, F_OUT_P), jnp.float32).at[0, :F_OUT].set(b2)
    return _final(agg2, gp, dinv, b2r)


# speedup vs baseline: 31.4979x; 1.1517x over previous
import functools

import jax
import jax.numpy as jnp
from jax import lax
from jax.experimental import pallas as pl
from jax.experimental.pallas import tpu as pltpu
from jax.experimental.pallas import tpu_sc as plsc

N = 50000
E = 1_600_000
F_IN = 3703
F_H = 16
F_OUT = 6
F_OUT_P = 8

NC, NS = 2, 16
NW = NC * NS

ROW_T = 512
GRID_N = 98
P = 50432

CH = 128
NCHUNKS = E // CH
CPT = 390
NEXTRA = NCHUNKS - CPT * NW
KG = 6
NGROUP = CPT // KG

_MESH = plsc.VectorSubcoreMesh(
    core_axis_name="c", subcore_axis_name="s", num_cores=NC, num_subcores=NS)


@functools.partial(
    pl.kernel,
    out_type=jax.ShapeDtypeStruct((NW, P), jnp.float32),
    mesh=_MESH,
    scratch_types=[
        pltpu.VMEM((CPT, CH), jnp.int32),
        pltpu.VMEM((CH,), jnp.int32),
        pltpu.VMEM((P,), jnp.float32),
    ],
    compiler_params=pltpu.CompilerParams(
        needs_layout_passes=False, use_tc_tiling_on_sc=False),
)
def _deg_kernel(edges_hbm, deg_hbm, idx_v, xidx_v, acc_v):
    wid = lax.axis_index("s") * NC + lax.axis_index("c")
    zero16 = jnp.zeros((16,), jnp.float32)
    ones16 = jnp.ones((16,), jnp.float32)

    def zbody(i, carry):
        acc_v[pl.ds(i * 16, 16)] = zero16
        return carry
    lax.fori_loop(0, P // 16, zbody, 0)

    pltpu.sync_copy(edges_hbm.at[1, pl.ds(wid * CPT, CPT)], idx_v)

    def sbody(c, carry):
        for j in range(CH // 16):
            idx16 = idx_v[c, pl.ds(j * 16, 16)]
            plsc.addupdate_scatter(acc_v, [idx16], ones16)
        return carry
    lax.fori_loop(0, CPT, sbody, 0)

    @pl.when(wid < NEXTRA)
    def _extra():
        pltpu.sync_copy(edges_hbm.at[1, CPT * NW + wid], xidx_v)
        for j in range(CH // 16):
            idx16 = xidx_v[pl.ds(j * 16, 16)]
            plsc.addupdate_scatter(acc_v, [idx16], ones16)

    pltpu.sync_copy(acc_v, deg_hbm.at[wid])


def _make_agg(width):
    @functools.partial(
        pl.kernel,
        out_type=jax.ShapeDtypeStruct((NC, P, width), jnp.float32),
        mesh=_MESH,
        scratch_types=[
            pltpu.VMEM((2, KG, CH), jnp.int32),
            pltpu.VMEM((2, KG, CH), jnp.int32),
            pltpu.VMEM((2, KG, CH, width), jnp.float32),
            pltpu.VMEM((1, CH), jnp.int32),
            pltpu.VMEM((1, CH), jnp.int32),
            pltpu.VMEM_SHARED((P, width), jnp.float32),
            pltpu.SemaphoreType.DMA,
            pltpu.SemaphoreType.DMA,
            pltpu.SemaphoreType.DMA,
        ],
        compiler_params=pltpu.CompilerParams(use_tc_tiling_on_sc=False),
    )
    def _agg(edges_hbm, feat_hbm, zeros_hbm, out_hbm,
             sidx_v, didx_v, rows_v, xsidx_v, xdidx_v, acc_s,
             gsem, ssem, isem):
        cid = lax.axis_index("c")
        sid = lax.axis_index("s")
        wid = sid * NC + cid
        cbase = wid * CPT
        rows_per = P // NS
        roff = pl.multiple_of(sid * rows_per, 8)

        pltpu.sync_copy(zeros_hbm.at[pl.ds(roff, rows_per)],
                        acc_s.at[pl.ds(roff, rows_per)])
        plsc.subcore_barrier()

        pltpu.async_copy(edges_hbm.at[0, pl.ds(cbase, KG)], sidx_v.at[0], isem)
        pltpu.async_copy(edges_hbm.at[1, pl.ds(cbase, KG)], didx_v.at[0], isem)

        def group(g, carry):
            b = lax.rem(g, 2)
            pltpu.make_async_copy(
                edges_hbm.at[0, pl.ds(0, KG)], sidx_v.at[b], isem).wait()
            pltpu.make_async_copy(
                edges_hbm.at[1, pl.ds(0, KG)], didx_v.at[b], isem).wait()
            descs = []
            for j in range(KG):
                descs.append(pltpu.async_copy(
                    feat_hbm.at[sidx_v.at[b, j]], rows_v.at[b, j], gsem))
            @pl.when(g + 1 < NGROUP)
            def _pfs():
                nb = cbase + (g + 1) * KG
                pltpu.async_copy(edges_hbm.at[0, pl.ds(nb, KG)],
                                 sidx_v.at[1 - b], isem)
            @pl.when(g > 0)
            def _drain():
                for j in range(KG):
                    pltpu.make_async_copy(
                        feat_hbm.at[pl.ds(0, CH)], rows_v.at[1 - b, j],
                        ssem).wait()
            @pl.when(g + 1 < NGROUP)
            def _pfd():
                nb = cbase + (g + 1) * KG
                pltpu.async_copy(edges_hbm.at[1, pl.ds(nb, KG)],
                                 didx_v.at[1 - b], isem)
            for j in range(KG):
                descs[j].wait()
            for j in range(KG):
                pltpu.async_copy(rows_v.at[b, j], acc_s.at[didx_v.at[b, j]],
                                 ssem, add=True)
            return carry
        lax.fori_loop(0, NGROUP, group, 0)

        lb = (NGROUP - 1) % 2
        for j in range(KG):
            pltpu.make_async_copy(
                feat_hbm.at[pl.ds(0, CH)], rows_v.at[lb, j], ssem).wait()

        @pl.when(wid < NEXTRA)
        def _extra():
            pltpu.sync_copy(edges_hbm.at[0, CPT * NW + wid], xsidx_v.at[0])
            pltpu.sync_copy(edges_hbm.at[1, CPT * NW + wid], xdidx_v.at[0])
            pltpu.async_copy(
                feat_hbm.at[xsidx_v.at[0]], rows_v.at[0, 0], gsem).wait()
            pltpu.sync_copy(rows_v.at[0, 0], acc_s.at[xdidx_v.at[0]],
                            add=True)

        plsc.subcore_barrier()
        pltpu.sync_copy(acc_s.at[pl.ds(roff, rows_per)],
                        out_hbm.at[cid, pl.ds(roff, rows_per)])
    return _agg


_agg16 = _make_agg(F_H)
_agg8 = _make_agg(F_OUT_P)


def _matmul_body(x_ref, w_ref, h_ref):
    h_ref[...] = jnp.dot(x_ref[...], w_ref[...],
                         preferred_element_type=jnp.float32)


def _matmul(x, W1):
    return pl.pallas_call(
        _matmul_body,
        grid=(GRID_N,),
        in_specs=[
            pl.BlockSpec((ROW_T, F_IN), lambda i: (i, 0)),
            pl.BlockSpec((F_IN, F_H), lambda i: (0, 0)),
        ],
        out_specs=pl.BlockSpec((ROW_T, F_H), lambda i: (i, 0)),
        out_shape=jax.ShapeDtypeStruct((N, F_H), jnp.float32),
    )(x, W1)


def _prep_body(h_ref, degp_ref, hp_ref, dinv_ref):
    deg = jnp.sum(degp_ref[...], axis=0) + 1.0
    dinv = lax.rsqrt(deg)
    hp_ref[...] = h_ref[...] * dinv[:, None]
    dinv_ref[...] = dinv[:, None]


def _prep(h, deg_parts):
    return pl.pallas_call(
        _prep_body,
        grid=(GRID_N,),
        in_specs=[
            pl.BlockSpec((ROW_T, F_H), lambda i: (i, 0)),
            pl.BlockSpec((NW, ROW_T), lambda i: (0, i)),
        ],
        out_specs=[
            pl.BlockSpec((ROW_T, F_H), lambda i: (i, 0)),
            pl.BlockSpec((ROW_T, 1), lambda i: (i, 0)),
        ],
        out_shape=[
            jax.ShapeDtypeStruct((N, F_H), jnp.float32),
            jax.ShapeDtypeStruct((N, 1), jnp.float32),
        ],
    )(h, deg_parts)


def _mid_body(agg_ref, hp_ref, dinv_ref, w2_ref, b1_ref, gp_ref):
    s = agg_ref[0] + agg_ref[1] + hp_ref[...]
    dinv = dinv_ref[...]
    out1 = s * dinv + b1_ref[...]
    r = jnp.maximum(out1, 0.0)
    g = jnp.dot(r, w2_ref[...], preferred_element_type=jnp.float32)
    gp_ref[...] = g * dinv


def _mid(agg1, hp, dinv, W2p, b1r):
    return pl.pallas_call(
        _mid_body,
        grid=(GRID_N,),
        in_specs=[
            pl.BlockSpec((NC, ROW_T, F_H), lambda i: (0, i, 0)),
            pl.BlockSpec((ROW_T, F_H), lambda i: (i, 0)),
            pl.BlockSpec((ROW_T, 1), lambda i: (i, 0)),
            pl.BlockSpec((F_H, F_OUT_P), lambda i: (0, 0)),
            pl.BlockSpec((1, F_H), lambda i: (0, 0)),
        ],
        out_specs=pl.BlockSpec((ROW_T, F_OUT_P), lambda i: (i, 0)),
        out_shape=jax.ShapeDtypeStruct((N, F_OUT_P), jnp.float32),
    )(agg1, hp, dinv, W2p, b1r)


def _final_body(agg_ref, gp_ref, dinv_ref, b2_ref, out_ref):
    s = agg_ref[0] + agg_ref[1] + gp_ref[...]
    z = s * dinv_ref[...] + b2_ref[...]
    col = lax.broadcasted_iota(jnp.int32, (ROW_T, F_OUT_P), 1)
    zm = jnp.where(col < F_OUT, z, -1e30)
    m = jnp.max(zm, axis=1, keepdims=True)
    ez = jnp.exp(zm - m)
    ssum = jnp.sum(ez, axis=1, keepdims=True)
    ls = (z - m) - jnp.log(ssum)
    out_ref[...] = ls[:, :F_OUT]


def _final(agg2, gp, dinv, b2r):
    return pl.pallas_call(
        _final_body,
        grid=(GRID_N,),
        in_specs=[
            pl.BlockSpec((NC, ROW_T, F_OUT_P), lambda i: (0, i, 0)),
            pl.BlockSpec((ROW_T, F_OUT_P), lambda i: (i, 0)),
            pl.BlockSpec((ROW_T, 1), lambda i: (i, 0)),
            pl.BlockSpec((1, F_OUT_P), lambda i: (0, 0)),
        ],
        out_specs=pl.BlockSpec((ROW_T, F_OUT), lambda i: (i, 0)),
        out_shape=jax.ShapeDtypeStruct((N, F_OUT), jnp.float32),
    )(agg2, gp, dinv, b2r)


def kernel(x, edge_index, W1, b1, W2, b2):
    edges3 = edge_index.reshape(2, NCHUNKS, CH)

    deg_parts = _deg_kernel(edges3)
    h = _matmul(x, W1)
    hp, dinv = _prep(h, deg_parts)

    z16 = jnp.zeros((P, F_H), jnp.float32)
    agg1 = _agg16(edges3, hp, z16)

    W2p = jnp.zeros((F_H, F_OUT_P), jnp.float32).at[:, :F_OUT].set(W2)
    b1r = b1.reshape(1, F_H)
    gp = _mid(agg1, hp, dinv, W2p, b1r)

    z8 = jnp.zeros((P, F_OUT_P), jnp.float32)
    agg2 = _agg8(edges3, gp, z8)

    b2r = jnp.zeros((1, F_OUT_P), jnp.float32).at[0, :F_OUT].set(b2)
    return _final(agg2, gp, dinv, b2r)


# speedup vs baseline: 62.4199x; 1.9817x over previous
import functools

import jax
import jax.numpy as jnp
from jax import lax
from jax.experimental import pallas as pl
from jax.experimental.pallas import tpu as pltpu
from jax.experimental.pallas import tpu_sc as plsc

N = 50000
E = 1_600_000
F_IN = 3703
F_H = 16
F_OUT = 6
F_OUT_P = 8

NC, NS = 2, 16
NW = NC * NS

ROW_T = 512
GRID_N = 98
ROW_B = 3200
GRID_B = 16
P = 50432

CH = 128
NCHUNKS = E // CH
CPT = 390
NEXTRA = NCHUNKS - CPT * NW
KG = 6
NGROUP = CPT // KG

_MESH = plsc.VectorSubcoreMesh(
    core_axis_name="c", subcore_axis_name="s", num_cores=NC, num_subcores=NS)


@functools.partial(
    pl.kernel,
    out_type=jax.ShapeDtypeStruct((NW, P), jnp.float32),
    mesh=_MESH,
    scratch_types=[
        pltpu.VMEM((CPT, CH), jnp.int32),
        pltpu.VMEM((CH,), jnp.int32),
        pltpu.VMEM((P,), jnp.float32),
    ],
    compiler_params=pltpu.CompilerParams(
        needs_layout_passes=False, use_tc_tiling_on_sc=False),
)
def _deg_kernel(edges_hbm, deg_hbm, idx_v, xidx_v, acc_v):
    wid = lax.axis_index("s") * NC + lax.axis_index("c")
    zero16 = jnp.zeros((16,), jnp.float32)
    ones16 = jnp.ones((16,), jnp.float32)

    def zbody(i, carry):
        acc_v[pl.ds(i * 16, 16)] = zero16
        return carry
    lax.fori_loop(0, P // 16, zbody, 0)

    pltpu.sync_copy(edges_hbm.at[1, pl.ds(wid * CPT, CPT)], idx_v)

    def sbody(c, carry):
        for j in range(CH // 16):
            idx16 = idx_v[c, pl.ds(j * 16, 16)]
            plsc.addupdate_scatter(acc_v, [idx16], ones16)
        return carry
    lax.fori_loop(0, CPT, sbody, 0)

    @pl.when(wid < NEXTRA)
    def _extra():
        pltpu.sync_copy(edges_hbm.at[1, CPT * NW + wid], xidx_v)
        for j in range(CH // 16):
            idx16 = xidx_v[pl.ds(j * 16, 16)]
            plsc.addupdate_scatter(acc_v, [idx16], ones16)

    pltpu.sync_copy(acc_v, deg_hbm.at[wid])


def _make_agg(width):
    @functools.partial(
        pl.kernel,
        out_type=jax.ShapeDtypeStruct((NC, P, width), jnp.float32),
        mesh=_MESH,
        scratch_types=[
            pltpu.VMEM((2, KG, CH), jnp.int32),
            pltpu.VMEM((2, KG, CH), jnp.int32),
            pltpu.VMEM((2, KG, CH, width), jnp.float32),
            pltpu.VMEM((1, CH), jnp.int32),
            pltpu.VMEM((1, CH), jnp.int32),
            pltpu.VMEM_SHARED((P, width), jnp.float32),
            pltpu.SemaphoreType.DMA,
            pltpu.SemaphoreType.DMA,
            pltpu.SemaphoreType.DMA,
        ],
        compiler_params=pltpu.CompilerParams(use_tc_tiling_on_sc=False),
    )
    def _agg(edges_hbm, feat_hbm, zeros_hbm, out_hbm,
             sidx_v, didx_v, rows_v, xsidx_v, xdidx_v, acc_s,
             gsem, ssem, isem):
        cid = lax.axis_index("c")
        sid = lax.axis_index("s")
        wid = sid * NC + cid
        cbase = wid * CPT
        rows_per = P // NS
        roff = pl.multiple_of(sid * rows_per, 8)

        pltpu.sync_copy(zeros_hbm.at[pl.ds(roff, rows_per)],
                        acc_s.at[pl.ds(roff, rows_per)])
        plsc.subcore_barrier()

        pltpu.async_copy(edges_hbm.at[0, pl.ds(cbase, KG)], sidx_v.at[0], isem)
        pltpu.async_copy(edges_hbm.at[1, pl.ds(cbase, KG)], didx_v.at[0], isem)

        def group(g, carry):
            b = lax.rem(g, 2)
            pltpu.make_async_copy(
                edges_hbm.at[0, pl.ds(0, KG)], sidx_v.at[b], isem).wait()
            pltpu.make_async_copy(
                edges_hbm.at[1, pl.ds(0, KG)], didx_v.at[b], isem).wait()
            descs = []
            for j in range(KG):
                descs.append(pltpu.async_copy(
                    feat_hbm.at[sidx_v.at[b, j]], rows_v.at[b, j], gsem))
            @pl.when(g + 1 < NGROUP)
            def _pfs():
                nb = cbase + (g + 1) * KG
                pltpu.async_copy(edges_hbm.at[0, pl.ds(nb, KG)],
                                 sidx_v.at[1 - b], isem)
            @pl.when(g > 0)
            def _drain():
                for j in range(KG):
                    pltpu.make_async_copy(
                        feat_hbm.at[pl.ds(0, CH)], rows_v.at[1 - b, j],
                        ssem).wait()
            @pl.when(g + 1 < NGROUP)
            def _pfd():
                nb = cbase + (g + 1) * KG
                pltpu.async_copy(edges_hbm.at[1, pl.ds(nb, KG)],
                                 didx_v.at[1 - b], isem)
            for j in range(KG):
                descs[j].wait()
            for j in range(KG):
                pltpu.async_copy(rows_v.at[b, j], acc_s.at[didx_v.at[b, j]],
                                 ssem, add=True)
            return carry
        lax.fori_loop(0, NGROUP, group, 0)

        lb = (NGROUP - 1) % 2
        for j in range(KG):
            pltpu.make_async_copy(
                feat_hbm.at[pl.ds(0, CH)], rows_v.at[lb, j], ssem).wait()

        @pl.when(wid < NEXTRA)
        def _extra():
            pltpu.sync_copy(edges_hbm.at[0, CPT * NW + wid], xsidx_v.at[0])
            pltpu.sync_copy(edges_hbm.at[1, CPT * NW + wid], xdidx_v.at[0])
            pltpu.async_copy(
                feat_hbm.at[xsidx_v.at[0]], rows_v.at[0, 0], gsem).wait()
            pltpu.sync_copy(rows_v.at[0, 0], acc_s.at[xdidx_v.at[0]],
                            add=True)

        plsc.subcore_barrier()
        pltpu.sync_copy(acc_s.at[pl.ds(roff, rows_per)],
                        out_hbm.at[cid, pl.ds(roff, rows_per)])
    return _agg


_agg16 = _make_agg(F_H)
_agg8 = _make_agg(F_OUT_P)


def _matmul_body(xt_ref, w_ref, h_ref):
    h_ref[...] = lax.dot_general(
        xt_ref[...], w_ref[...], (((0,), (0,)), ((), ())),
        preferred_element_type=jnp.float32)


def _matmul(xt, W1):
    return pl.pallas_call(
        _matmul_body,
        grid=(GRID_N,),
        in_specs=[
            pl.BlockSpec((F_IN, ROW_T), lambda i: (0, i)),
            pl.BlockSpec((F_IN, F_H), lambda i: (0, 0)),
        ],
        out_specs=pl.BlockSpec((ROW_T, F_H), lambda i: (i, 0)),
        out_shape=jax.ShapeDtypeStruct((N, F_H), jnp.float32),
    )(xt, W1)


def _prep_body(h_ref, degp_ref, hp_ref, dinv_ref):
    deg = jnp.sum(degp_ref[...], axis=0) + 1.0
    dinv = lax.rsqrt(deg)
    hp_ref[...] = h_ref[...] * dinv[:, None]
    dinv_ref[...] = dinv[:, None]


def _prep(h, deg_parts):
    return pl.pallas_call(
        _prep_body,
        grid=(GRID_B,),
        in_specs=[
            pl.BlockSpec((ROW_B, F_H), lambda i: (i, 0)),
            pl.BlockSpec((NW, ROW_B), lambda i: (0, i)),
        ],
        out_specs=[
            pl.BlockSpec((ROW_B, F_H), lambda i: (i, 0)),
            pl.BlockSpec((ROW_B, 1), lambda i: (i, 0)),
        ],
        out_shape=[
            jax.ShapeDtypeStruct((N, F_H), jnp.float32),
            jax.ShapeDtypeStruct((N, 1), jnp.float32),
        ],
    )(h, deg_parts)


def _mid_body(agg_ref, hp_ref, dinv_ref, w2_ref, b1_ref, gp_ref):
    s = agg_ref[0] + agg_ref[1] + hp_ref[...]
    dinv = dinv_ref[...]
    out1 = s * dinv + b1_ref[...]
    r = jnp.maximum(out1, 0.0)
    g = jnp.dot(r, w2_ref[...], preferred_element_type=jnp.float32)
    gp_ref[...] = g * dinv


def _mid(agg1, hp, dinv, W2p, b1r):
    return pl.pallas_call(
        _mid_body,
        grid=(GRID_B,),
        in_specs=[
            pl.BlockSpec((NC, ROW_B, F_H), lambda i: (0, i, 0)),
            pl.BlockSpec((ROW_B, F_H), lambda i: (i, 0)),
            pl.BlockSpec((ROW_B, 1), lambda i: (i, 0)),
            pl.BlockSpec((F_H, F_OUT_P), lambda i: (0, 0)),
            pl.BlockSpec((1, F_H), lambda i: (0, 0)),
        ],
        out_specs=pl.BlockSpec((ROW_B, F_OUT_P), lambda i: (i, 0)),
        out_shape=jax.ShapeDtypeStruct((N, F_OUT_P), jnp.float32),
    )(agg1, hp, dinv, W2p, b1r)


def _final_body(agg_ref, gp_ref, dinv_ref, b2_ref, out_ref):
    s = agg_ref[0] + agg_ref[1] + gp_ref[...]
    z = s * dinv_ref[...] + b2_ref[...]
    col = lax.broadcasted_iota(jnp.int32, (ROW_B, F_OUT_P), 1)
    zm = jnp.where(col < F_OUT, z, -1e30)
    m = jnp.max(zm, axis=1, keepdims=True)
    ez = jnp.exp(zm - m)
    ssum = jnp.sum(ez, axis=1, keepdims=True)
    ls = (z - m) - jnp.log(ssum)
    out_ref[...] = ls[:, :F_OUT]


def _final(agg2, gp, dinv, b2r):
    return pl.pallas_call(
        _final_body,
        grid=(GRID_B,),
        in_specs=[
            pl.BlockSpec((NC, ROW_B, F_OUT_P), lambda i: (0, i, 0)),
            pl.BlockSpec((ROW_B, F_OUT_P), lambda i: (i, 0)),
            pl.BlockSpec((ROW_B, 1), lambda i: (i, 0)),
            pl.BlockSpec((1, F_OUT_P), lambda i: (0, 0)),
        ],
        out_specs=pl.BlockSpec((ROW_B, F_OUT), lambda i: (i, 0)),
        out_shape=jax.ShapeDtypeStruct((N, F_OUT), jnp.float32),
    )(agg2, gp, dinv, b2r)


def kernel(x, edge_index, W1, b1, W2, b2):
    edges3 = edge_index.reshape(2, NCHUNKS, CH)

    deg_parts = _deg_kernel(edges3)
    h = _matmul(x.T, W1)
    hp, dinv = _prep(h, deg_parts)

    z16 = jnp.zeros((P, F_H), jnp.float32)
    agg1 = _agg16(edges3, hp, z16)

    W2p = jnp.zeros((F_H, F_OUT_P), jnp.float32).at[:, :F_OUT].set(W2)
    b1r = b1.reshape(1, F_H)
    gp = _mid(agg1, hp, dinv, W2p, b1r)

    z8 = jnp.zeros((P, F_OUT_P), jnp.float32)
    agg2 = _agg8(edges3, gp, z8)

    b2r = jnp.zeros((1, F_OUT_P), jnp.float32).at[0, :F_OUT].set(b2)
    return _final(agg2, gp, dinv, b2r)


# speedup vs baseline: 70.4300x; 1.1283x over previous
import functools

import jax
import jax.numpy as jnp
from jax import lax
from jax.experimental import pallas as pl
from jax.experimental.pallas import tpu as pltpu
from jax.experimental.pallas import tpu_sc as plsc

N = 50000
E = 1_600_000
F_IN = 3703
F_H = 16
F_OUT = 6
F_OUT_P = 8

NC, NS = 2, 16
NW = NC * NS

ROW_T = 1024
GRID_N = 49
ROW_B = 3200
GRID_B = 16
P = 50432

CH = 128
NCHUNKS = E // CH
CPT = 390
NEXTRA = NCHUNKS - CPT * NW
KG = 10
NGROUP = CPT // KG
DGK = 30

_MESH = plsc.VectorSubcoreMesh(
    core_axis_name="c", subcore_axis_name="s", num_cores=NC, num_subcores=NS)


@functools.partial(
    pl.kernel,
    out_type=jax.ShapeDtypeStruct((NW, P), jnp.float32),
    mesh=_MESH,
    scratch_types=[
        pltpu.VMEM((DGK, 2, CH), jnp.int32),
        pltpu.VMEM((P,), jnp.float32),
    ],
    compiler_params=pltpu.CompilerParams(
        needs_layout_passes=False, use_tc_tiling_on_sc=False),
)
def _deg_kernel(edges_hbm, deg_hbm, ebuf_v, acc_v):
    wid = lax.axis_index("s") * NC + lax.axis_index("c")
    zero16 = jnp.zeros((16,), jnp.float32)
    ones16 = jnp.ones((16,), jnp.float32)

    def zbody(i, carry):
        acc_v[pl.ds(i * 16, 16)] = zero16
        return carry
    lax.fori_loop(0, P // 16, zbody, 0)

    def gbody(g, carry):
        pltpu.sync_copy(edges_hbm.at[pl.ds(wid * CPT + g * DGK, DGK)], ebuf_v)

        def sbody(c, carry2):
            for j in range(CH // 16):
                idx16 = ebuf_v[c, 1, pl.ds(j * 16, 16)]
                plsc.addupdate_scatter(acc_v, [idx16], ones16)
            return carry2
        lax.fori_loop(0, DGK, sbody, 0)
        return carry
    lax.fori_loop(0, CPT // DGK, gbody, 0)

    @pl.when(wid < NEXTRA)
    def _extra():
        pltpu.sync_copy(edges_hbm.at[pl.ds(CPT * NW + wid, 1)],
                        ebuf_v.at[pl.ds(0, 1)])
        for j in range(CH // 16):
            idx16 = ebuf_v[0, 1, pl.ds(j * 16, 16)]
            plsc.addupdate_scatter(acc_v, [idx16], ones16)

    pltpu.sync_copy(acc_v, deg_hbm.at[wid])


def _make_agg(width):
    @functools.partial(
        pl.kernel,
        out_type=jax.ShapeDtypeStruct((NC, P, width), jnp.float32),
        mesh=_MESH,
        scratch_types=[
            pltpu.VMEM((2, KG, 2, CH), jnp.int32),
            pltpu.VMEM((2, KG * CH, width), jnp.float32),
            pltpu.VMEM((1, 2, CH), jnp.int32),
            pltpu.VMEM_SHARED((P, width), jnp.float32),
            pltpu.SemaphoreType.DMA,
            pltpu.SemaphoreType.DMA,
            pltpu.SemaphoreType.DMA,
        ],
        compiler_params=pltpu.CompilerParams(use_tc_tiling_on_sc=False),
    )
    def _agg(edges_hbm, feat_hbm, zeros_hbm, out_hbm,
             ebuf_v, rows_v, xebuf_v, acc_s,
             gsem, ssem, isem):
        cid = lax.axis_index("c")
        sid = lax.axis_index("s")
        wid = sid * NC + cid
        cbase = wid * CPT
        rows_per = P // NS
        roff = pl.multiple_of(sid * rows_per, 8)

        pltpu.sync_copy(zeros_hbm.at[pl.ds(roff, rows_per)],
                        acc_s.at[pl.ds(roff, rows_per)])
        plsc.subcore_barrier()

        pltpu.async_copy(edges_hbm.at[pl.ds(cbase, KG)], ebuf_v.at[0], isem)

        def group(g, carry):
            b = lax.rem(g, 2)
            pltpu.make_async_copy(
                edges_hbm.at[pl.ds(0, KG)], ebuf_v.at[b], isem).wait()
            for j in range(KG):
                pltpu.async_copy(
                    feat_hbm.at[ebuf_v.at[b, j, 0]],
                    rows_v.at[b, pl.ds(j * CH, CH)], gsem)
            @pl.when(g > 0)
            def _drain():
                pltpu.make_async_copy(
                    feat_hbm.at[pl.ds(0, KG * CH)], rows_v.at[1 - b],
                    ssem).wait()
            @pl.when(g + 1 < NGROUP)
            def _pf():
                nb = cbase + (g + 1) * KG
                pltpu.async_copy(edges_hbm.at[pl.ds(nb, KG)],
                                 ebuf_v.at[1 - b], isem)
            pltpu.make_async_copy(
                feat_hbm.at[pl.ds(0, KG * CH)], rows_v.at[b], gsem).wait()
            for j in range(KG):
                pltpu.async_copy(rows_v.at[b, pl.ds(j * CH, CH)],
                                 acc_s.at[ebuf_v.at[b, j, 1]],
                                 ssem, add=True)
            return carry
        lax.fori_loop(0, NGROUP, group, 0)

        lb = (NGROUP - 1) % 2
        pltpu.make_async_copy(
            feat_hbm.at[pl.ds(0, KG * CH)], rows_v.at[lb], ssem).wait()

        @pl.when(wid < NEXTRA)
        def _extra():
            pltpu.sync_copy(edges_hbm.at[pl.ds(CPT * NW + wid, 1)], xebuf_v)
            pltpu.async_copy(
                feat_hbm.at[xebuf_v.at[0, 0]], rows_v.at[0, pl.ds(0, CH)],
                gsem).wait()
            pltpu.sync_copy(rows_v.at[0, pl.ds(0, CH)],
                            acc_s.at[xebuf_v.at[0, 1]],
                            add=True)

        plsc.subcore_barrier()
        pltpu.sync_copy(acc_s.at[pl.ds(roff, rows_per)],
                        out_hbm.at[cid, pl.ds(roff, rows_per)])
    return _agg


_agg16 = _make_agg(F_H)
_agg8 = _make_agg(F_OUT_P)


def _matmul_body(xt_ref, w_ref, h_ref):
    h_ref[...] = lax.dot_general(
        xt_ref[...], w_ref[...], (((0,), (0,)), ((), ())),
        preferred_element_type=jnp.float32)


def _matmul(xt, W1):
    return pl.pallas_call(
        _matmul_body,
        grid=(GRID_N,),
        in_specs=[
            pl.BlockSpec((F_IN, ROW_T), lambda i: (0, i)),
            pl.BlockSpec((F_IN, F_H), lambda i: (0, 0)),
        ],
        out_specs=pl.BlockSpec((ROW_T, F_H), lambda i: (i, 0)),
        out_shape=jax.ShapeDtypeStruct((N, F_H), jnp.float32),
    )(xt, W1)


def _prep_body(h_ref, degp_ref, hp_ref, dinv_ref):
    deg = jnp.sum(degp_ref[...], axis=0) + 1.0
    dinv = lax.rsqrt(deg)
    hp_ref[...] = h_ref[...] * dinv[:, None]
    dinv_ref[...] = dinv[:, None]


def _prep(h, deg_parts):
    return pl.pallas_call(
        _prep_body,
        grid=(GRID_B,),
        in_specs=[
            pl.BlockSpec((ROW_B, F_H), lambda i: (i, 0)),
            pl.BlockSpec((NW, ROW_B), lambda i: (0, i)),
        ],
        out_specs=[
            pl.BlockSpec((ROW_B, F_H), lambda i: (i, 0)),
            pl.BlockSpec((ROW_B, 1), lambda i: (i, 0)),
        ],
        out_shape=[
            jax.ShapeDtypeStruct((N, F_H), jnp.float32),
            jax.ShapeDtypeStruct((N, 1), jnp.float32),
        ],
    )(h, deg_parts)


def _mid_body(agg_ref, hp_ref, dinv_ref, w2_ref, b1_ref, gp_ref):
    s = agg_ref[0] + agg_ref[1] + hp_ref[...]
    dinv = dinv_ref[...]
    out1 = s * dinv + b1_ref[...]
    r = jnp.maximum(out1, 0.0)
    g = jnp.dot(r, w2_ref[...], preferred_element_type=jnp.float32)
    gp_ref[...] = g * dinv


def _mid(agg1, hp, dinv, W2p, b1r):
    return pl.pallas_call(
        _mid_body,
        grid=(GRID_B,),
        in_specs=[
            pl.BlockSpec((NC, ROW_B, F_H), lambda i: (0, i, 0)),
            pl.BlockSpec((ROW_B, F_H), lambda i: (i, 0)),
            pl.BlockSpec((ROW_B, 1), lambda i: (i, 0)),
            pl.BlockSpec((F_H, F_OUT_P), lambda i: (0, 0)),
            pl.BlockSpec((1, F_H), lambda i: (0, 0)),
        ],
        out_specs=pl.BlockSpec((ROW_B, F_OUT_P), lambda i: (i, 0)),
        out_shape=jax.ShapeDtypeStruct((N, F_OUT_P), jnp.float32),
    )(agg1, hp, dinv, W2p, b1r)


def _final_body(agg_ref, gp_ref, dinv_ref, b2_ref, out_ref):
    s = agg_ref[0] + agg_ref[1] + gp_ref[...]
    z = s * dinv_ref[...] + b2_ref[...]
    col = lax.broadcasted_iota(jnp.int32, (ROW_B, F_OUT_P), 1)
    zm = jnp.where(col < F_OUT, z, -1e30)
    m = jnp.max(zm, axis=1, keepdims=True)
    ez = jnp.exp(zm - m)
    ssum = jnp.sum(ez, axis=1, keepdims=True)
    ls = (z - m) - jnp.log(ssum)
    out_ref[...] = ls[:, :F_OUT]


def _final(agg2, gp, dinv, b2r):
    return pl.pallas_call(
        _final_body,
        grid=(GRID_B,),
        in_specs=[
            pl.BlockSpec((NC, ROW_B, F_OUT_P), lambda i: (0, i, 0)),
            pl.BlockSpec((ROW_B, F_OUT_P), lambda i: (i, 0)),
            pl.BlockSpec((ROW_B, 1), lambda i: (i, 0)),
            pl.BlockSpec((1, F_OUT_P), lambda i: (0, 0)),
        ],
        out_specs=pl.BlockSpec((ROW_B, F_OUT), lambda i: (i, 0)),
        out_shape=jax.ShapeDtypeStruct((N, F_OUT), jnp.float32),
    )(agg2, gp, dinv, b2r)


def kernel(x, edge_index, W1, b1, W2, b2):
    edges3 = jnp.transpose(edge_index.reshape(2, NCHUNKS, CH), (1, 0, 2))

    deg_parts = _deg_kernel(edges3)
    h = _matmul(x.T, W1)
    hp, dinv = _prep(h, deg_parts)

    z16 = jnp.zeros((P, F_H), jnp.float32)
    agg1 = _agg16(edges3, hp, z16)

    W2p = jnp.zeros((F_H, F_OUT_P), jnp.float32).at[:, :F_OUT].set(W2)
    b1r = b1.reshape(1, F_H)
    gp = _mid(agg1, hp, dinv, W2p, b1r)

    z8 = jnp.zeros((P, F_OUT_P), jnp.float32)
    agg2 = _agg8(edges3, gp, z8)

    b2r = jnp.zeros((1, F_OUT_P), jnp.float32).at[0, :F_OUT].set(b2)
    return _final(agg2, gp, dinv, b2r)


# speedup vs baseline: 72.4600x; 1.0288x over previous
import functools

import jax
import jax.numpy as jnp
from jax import lax
from jax.experimental import pallas as pl
from jax.experimental.pallas import tpu as pltpu
from jax.experimental.pallas import tpu_sc as plsc

N = 50000
E = 1_600_000
F_IN = 3703
F_H = 16
F_OUT = 6
F_OUT_P = 8

NC, NS = 2, 16
NW = NC * NS

ROW_T = 1280
GRID_N = 40
ROW_B = 6400
GRID_B = 8
P = 50432

CH = 128
NCHUNKS = E // CH
CPT = 390
NEXTRA = NCHUNKS - CPT * NW
KG = 10
NGROUP = CPT // KG
DGK = 30

_MESH = plsc.VectorSubcoreMesh(
    core_axis_name="c", subcore_axis_name="s", num_cores=NC, num_subcores=NS)


@functools.partial(
    pl.kernel,
    out_type=jax.ShapeDtypeStruct((NW, P), jnp.float32),
    mesh=_MESH,
    scratch_types=[
        pltpu.VMEM((DGK, 2, CH), jnp.int32),
        pltpu.VMEM((P,), jnp.float32),
    ],
    compiler_params=pltpu.CompilerParams(
        needs_layout_passes=False, use_tc_tiling_on_sc=False),
)
def _deg_kernel(edges_hbm, deg_hbm, ebuf_v, acc_v):
    wid = lax.axis_index("s") * NC + lax.axis_index("c")
    zero16 = jnp.zeros((16,), jnp.float32)
    ones16 = jnp.ones((16,), jnp.float32)

    def zbody(i, carry):
        acc_v[pl.ds(i * 16, 16)] = zero16
        return carry
    lax.fori_loop(0, P // 16, zbody, 0)

    def gbody(g, carry):
        pltpu.sync_copy(edges_hbm.at[pl.ds(wid * CPT + g * DGK, DGK)], ebuf_v)

        def sbody(c, carry2):
            for j in range(CH // 16):
                idx16 = ebuf_v[c, 1, pl.ds(j * 16, 16)]
                plsc.addupdate_scatter(acc_v, [idx16], ones16)
            return carry2
        lax.fori_loop(0, DGK, sbody, 0)
        return carry
    lax.fori_loop(0, CPT // DGK, gbody, 0)

    @pl.when(wid < NEXTRA)
    def _extra():
        pltpu.sync_copy(edges_hbm.at[pl.ds(CPT * NW + wid, 1)],
                        ebuf_v.at[pl.ds(0, 1)])
        for j in range(CH // 16):
            idx16 = ebuf_v[0, 1, pl.ds(j * 16, 16)]
            plsc.addupdate_scatter(acc_v, [idx16], ones16)

    pltpu.sync_copy(acc_v, deg_hbm.at[wid])


def _make_agg(width):
    @functools.partial(
        pl.kernel,
        out_type=jax.ShapeDtypeStruct((NC, P, width), jnp.float32),
        mesh=_MESH,
        scratch_types=[
            pltpu.VMEM((2, KG, 2, CH), jnp.int32),
            pltpu.VMEM((2, KG * CH, width), jnp.float32),
            pltpu.VMEM((1, 2, CH), jnp.int32),
            pltpu.VMEM_SHARED((P, width), jnp.float32),
            pltpu.SemaphoreType.DMA,
            pltpu.SemaphoreType.DMA,
            pltpu.SemaphoreType.DMA,
        ],
        compiler_params=pltpu.CompilerParams(use_tc_tiling_on_sc=False),
    )
    def _agg(edges_hbm, feat_hbm, zeros_hbm, out_hbm,
             ebuf_v, rows_v, xebuf_v, acc_s,
             gsem, ssem, isem):
        cid = lax.axis_index("c")
        sid = lax.axis_index("s")
        wid = sid * NC + cid
        cbase = wid * CPT
        rows_per = P // NS
        roff = pl.multiple_of(sid * rows_per, 8)

        pltpu.sync_copy(zeros_hbm.at[pl.ds(roff, rows_per)],
                        acc_s.at[pl.ds(roff, rows_per)])
        plsc.subcore_barrier()

        pltpu.async_copy(edges_hbm.at[pl.ds(cbase, KG)], ebuf_v.at[0], isem)

        def group(g, carry):
            b = lax.rem(g, 2)
            pltpu.make_async_copy(
                edges_hbm.at[pl.ds(0, KG)], ebuf_v.at[b], isem).wait()
            for j in range(KG):
                pltpu.async_copy(
                    feat_hbm.at[ebuf_v.at[b, j, 0]],
                    rows_v.at[b, pl.ds(j * CH, CH)], gsem)
            @pl.when(g > 0)
            def _drain():
                pltpu.make_async_copy(
                    feat_hbm.at[pl.ds(0, KG * CH)], rows_v.at[1 - b],
                    ssem).wait()
            @pl.when(g + 1 < NGROUP)
            def _pf():
                nb = cbase + (g + 1) * KG
                pltpu.async_copy(edges_hbm.at[pl.ds(nb, KG)],
                                 ebuf_v.at[1 - b], isem)
            pltpu.make_async_copy(
                feat_hbm.at[pl.ds(0, KG * CH)], rows_v.at[b], gsem).wait()
            for j in range(KG):
                pltpu.async_copy(rows_v.at[b, pl.ds(j * CH, CH)],
                                 acc_s.at[ebuf_v.at[b, j, 1]],
                                 ssem, add=True)
            return carry
        lax.fori_loop(0, NGROUP, group, 0)

        lb = (NGROUP - 1) % 2
        pltpu.make_async_copy(
            feat_hbm.at[pl.ds(0, KG * CH)], rows_v.at[lb], ssem).wait()

        @pl.when(wid < NEXTRA)
        def _extra():
            pltpu.sync_copy(edges_hbm.at[pl.ds(CPT * NW + wid, 1)], xebuf_v)
            pltpu.async_copy(
                feat_hbm.at[xebuf_v.at[0, 0]], rows_v.at[0, pl.ds(0, CH)],
                gsem).wait()
            pltpu.sync_copy(rows_v.at[0, pl.ds(0, CH)],
                            acc_s.at[xebuf_v.at[0, 1]],
                            add=True)

        plsc.subcore_barrier()
        pltpu.sync_copy(acc_s.at[pl.ds(roff, rows_per)],
                        out_hbm.at[cid, pl.ds(roff, rows_per)])
    return _agg


_agg16 = _make_agg(F_H)
_agg8 = _make_agg(F_OUT_P)


def _matmul_body(xt_ref, w_ref, h_ref):
    h_ref[...] = lax.dot_general(
        xt_ref[...], w_ref[...], (((0,), (0,)), ((), ())),
        preferred_element_type=jnp.float32)


def _matmul(xt, W1):
    return pl.pallas_call(
        _matmul_body,
        grid=(GRID_N,),
        in_specs=[
            pl.BlockSpec((F_IN, ROW_T), lambda i: (0, i)),
            pl.BlockSpec((F_IN, F_H), lambda i: (0, 0)),
        ],
        out_specs=pl.BlockSpec((ROW_T, F_H), lambda i: (i, 0)),
        out_shape=jax.ShapeDtypeStruct((N, F_H), jnp.float32),
    )(xt, W1)


def _prep_body(h_ref, degp_ref, hp_ref, dinv_ref):
    deg = jnp.sum(degp_ref[...], axis=0) + 1.0
    dinv = lax.rsqrt(deg)
    hp_ref[...] = h_ref[...] * dinv[:, None]
    dinv_ref[...] = dinv[:, None]


def _prep(h, deg_parts):
    return pl.pallas_call(
        _prep_body,
        grid=(GRID_B,),
        in_specs=[
            pl.BlockSpec((ROW_B, F_H), lambda i: (i, 0)),
            pl.BlockSpec((NW, ROW_B), lambda i: (0, i)),
        ],
        out_specs=[
            pl.BlockSpec((ROW_B, F_H), lambda i: (i, 0)),
            pl.BlockSpec((ROW_B, 1), lambda i: (i, 0)),
        ],
        out_shape=[
            jax.ShapeDtypeStruct((N, F_H), jnp.float32),
            jax.ShapeDtypeStruct((N, 1), jnp.float32),
        ],
    )(h, deg_parts)


def _mid_body(agg_ref, hp_ref, dinv_ref, w2_ref, b1_ref, gp_ref):
    s = agg_ref[0] + agg_ref[1] + hp_ref[...]
    dinv = dinv_ref[...]
    out1 = s * dinv + b1_ref[...]
    r = jnp.maximum(out1, 0.0)
    g = jnp.dot(r, w2_ref[...], preferred_element_type=jnp.float32)
    gp_ref[...] = g * dinv


def _mid(agg1, hp, dinv, W2p, b1r):
    return pl.pallas_call(
        _mid_body,
        grid=(GRID_B,),
        in_specs=[
            pl.BlockSpec((NC, ROW_B, F_H), lambda i: (0, i, 0)),
            pl.BlockSpec((ROW_B, F_H), lambda i: (i, 0)),
            pl.BlockSpec((ROW_B, 1), lambda i: (i, 0)),
            pl.BlockSpec((F_H, F_OUT_P), lambda i: (0, 0)),
            pl.BlockSpec((1, F_H), lambda i: (0, 0)),
        ],
        out_specs=pl.BlockSpec((ROW_B, F_OUT_P), lambda i: (i, 0)),
        out_shape=jax.ShapeDtypeStruct((N, F_OUT_P), jnp.float32),
    )(agg1, hp, dinv, W2p, b1r)


def _final_body(agg_ref, gp_ref, dinv_ref, b2_ref, out_ref):
    s = agg_ref[0] + agg_ref[1] + gp_ref[...]
    z = s * dinv_ref[...] + b2_ref[...]
    col = lax.broadcasted_iota(jnp.int32, (ROW_B, F_OUT_P), 1)
    zm = jnp.where(col < F_OUT, z, -1e30)
    m = jnp.max(zm, axis=1, keepdims=True)
    ez = jnp.exp(zm - m)
    ssum = jnp.sum(ez, axis=1, keepdims=True)
    ls = (z - m) - jnp.log(ssum)
    out_ref[...] = ls[:, :F_OUT].T


def _final(agg2, gp, dinv, b2r):
    return pl.pallas_call(
        _final_body,
        grid=(GRID_B,),
        in_specs=[
            pl.BlockSpec((NC, ROW_B, F_OUT_P), lambda i: (0, i, 0)),
            pl.BlockSpec((ROW_B, F_OUT_P), lambda i: (i, 0)),
            pl.BlockSpec((ROW_B, 1), lambda i: (i, 0)),
            pl.BlockSpec((1, F_OUT_P), lambda i: (0, 0)),
        ],
        out_specs=pl.BlockSpec((F_OUT, ROW_B), lambda i: (0, i)),
        out_shape=jax.ShapeDtypeStruct((F_OUT, N), jnp.float32),
    )(agg2, gp, dinv, b2r)


def kernel(x, edge_index, W1, b1, W2, b2):
    edges3 = jnp.transpose(edge_index.reshape(2, NCHUNKS, CH), (1, 0, 2))

    deg_parts = _deg_kernel(edges3)
    h = _matmul(x.T, W1)
    hp, dinv = _prep(h, deg_parts)

    z16 = jnp.zeros((P, F_H), jnp.float32)
    agg1 = _agg16(edges3, hp, z16)

    W2p = jnp.zeros((F_H, F_OUT_P), jnp.float32).at[:, :F_OUT].set(W2)
    b1r = b1.reshape(1, F_H)
    gp = _mid(agg1, hp, dinv, W2p, b1r)

    z8 = jnp.zeros((P, F_OUT_P), jnp.float32)
    agg2 = _agg8(edges3, gp, z8)

    b2r = jnp.zeros((1, F_OUT_P), jnp.float32).at[0, :F_OUT].set(b2)
    return _final(agg2, gp, dinv, b2r).T


# speedup vs baseline: 74.2405x; 1.0246x over previous
import functools

import jax
import jax.numpy as jnp
from jax import lax
from jax.experimental import pallas as pl
from jax.experimental.pallas import tpu as pltpu
from jax.experimental.pallas import tpu_sc as plsc

N = 50000
E = 1_600_000
F_IN = 3703
F_H = 16
F_OUT = 6
F_OUT_P = 8

NC, NS = 2, 16
NW = NC * NS

ROW_T = 1280
GRID_N = 40
ROW_B = 6400
GRID_B = 8
P = 50432

CH = 128
NCHUNKS = E // CH
CPT = 390
NEXTRA = NCHUNKS - CPT * NW
KG = 10
NGROUP = CPT // KG
DGK = 30

_MESH = plsc.VectorSubcoreMesh(
    core_axis_name="c", subcore_axis_name="s", num_cores=NC, num_subcores=NS)


@functools.partial(
    pl.kernel,
    out_type=jax.ShapeDtypeStruct((NW, P), jnp.float32),
    mesh=_MESH,
    scratch_types=[
        pltpu.VMEM((DGK, 2, CH), jnp.int32),
        pltpu.VMEM((P,), jnp.float32),
    ],
    compiler_params=pltpu.CompilerParams(
        needs_layout_passes=False, use_tc_tiling_on_sc=False),
)
def _deg_kernel(edges_hbm, deg_hbm, ebuf_v, acc_v):
    wid = lax.axis_index("s") * NC + lax.axis_index("c")
    zero16 = jnp.zeros((16,), jnp.float32)
    ones16 = jnp.ones((16,), jnp.float32)

    def zbody(i, carry):
        acc_v[pl.ds(i * 16, 16)] = zero16
        return carry
    lax.fori_loop(0, P // 16, zbody, 0)

    def gbody(g, carry):
        pltpu.sync_copy(edges_hbm.at[pl.ds(wid * CPT + g * DGK, DGK)], ebuf_v)

        def sbody(c, carry2):
            for j in range(CH // 16):
                idx16 = ebuf_v[c, 1, pl.ds(j * 16, 16)]
                plsc.addupdate_scatter(acc_v, [idx16], ones16)
            return carry2
        lax.fori_loop(0, DGK, sbody, 0)
        return carry
    lax.fori_loop(0, CPT // DGK, gbody, 0)

    @pl.when(wid < NEXTRA)
    def _extra():
        pltpu.sync_copy(edges_hbm.at[pl.ds(CPT * NW + wid, 1)],
                        ebuf_v.at[pl.ds(0, 1)])
        for j in range(CH // 16):
            idx16 = ebuf_v[0, 1, pl.ds(j * 16, 16)]
            plsc.addupdate_scatter(acc_v, [idx16], ones16)

    pltpu.sync_copy(acc_v, deg_hbm.at[wid])


def _make_agg(width):
    @functools.partial(
        pl.kernel,
        out_type=jax.ShapeDtypeStruct((NC, P, width), jnp.float32),
        mesh=_MESH,
        scratch_types=[
            pltpu.VMEM((2, KG, 2, CH), jnp.int32),
            pltpu.VMEM((2, KG * CH, width), jnp.float32),
            pltpu.VMEM((1, 2, CH), jnp.int32),
            pltpu.VMEM_SHARED((P, width), jnp.float32),
            pltpu.SemaphoreType.DMA,
            pltpu.SemaphoreType.DMA,
            pltpu.SemaphoreType.DMA,
        ],
        compiler_params=pltpu.CompilerParams(use_tc_tiling_on_sc=False),
    )
    def _agg(edges_hbm, feat_hbm, zeros_hbm, out_hbm,
             ebuf_v, rows_v, xebuf_v, acc_s,
             gsem, ssem, isem):
        cid = lax.axis_index("c")
        sid = lax.axis_index("s")
        wid = sid * NC + cid
        cbase = wid * CPT
        rows_per = P // NS
        roff = pl.multiple_of(sid * rows_per, 8)

        pltpu.sync_copy(zeros_hbm.at[pl.ds(roff, rows_per)],
                        acc_s.at[pl.ds(roff, rows_per)])
        plsc.subcore_barrier()

        pltpu.async_copy(edges_hbm.at[pl.ds(cbase, KG)], ebuf_v.at[0], isem)

        def group(g, carry):
            b = lax.rem(g, 2)
            pltpu.make_async_copy(
                edges_hbm.at[pl.ds(0, KG)], ebuf_v.at[b], isem).wait()
            for j in range(KG):
                pltpu.async_copy(
                    feat_hbm.at[ebuf_v.at[b, j, 0]],
                    rows_v.at[b, pl.ds(j * CH, CH)], gsem)
            @pl.when(g > 0)
            def _drain():
                pltpu.make_async_copy(
                    feat_hbm.at[pl.ds(0, KG * CH)], rows_v.at[1 - b],
                    ssem).wait()
            @pl.when(g + 1 < NGROUP)
            def _pf():
                nb = cbase + (g + 1) * KG
                pltpu.async_copy(edges_hbm.at[pl.ds(nb, KG)],
                                 ebuf_v.at[1 - b], isem)
            pltpu.make_async_copy(
                feat_hbm.at[pl.ds(0, KG * CH)], rows_v.at[b], gsem).wait()
            for j in range(KG):
                pltpu.async_copy(rows_v.at[b, pl.ds(j * CH, CH)],
                                 acc_s.at[ebuf_v.at[b, j, 1]],
                                 ssem, add=True)
            return carry
        lax.fori_loop(0, NGROUP, group, 0)

        lb = (NGROUP - 1) % 2
        pltpu.make_async_copy(
            feat_hbm.at[pl.ds(0, KG * CH)], rows_v.at[lb], ssem).wait()

        @pl.when(wid < NEXTRA)
        def _extra():
            pltpu.sync_copy(edges_hbm.at[pl.ds(CPT * NW + wid, 1)], xebuf_v)
            pltpu.async_copy(
                feat_hbm.at[xebuf_v.at[0, 0]], rows_v.at[0, pl.ds(0, CH)],
                gsem).wait()
            pltpu.sync_copy(rows_v.at[0, pl.ds(0, CH)],
                            acc_s.at[xebuf_v.at[0, 1]],
                            add=True)

        plsc.subcore_barrier()
        pltpu.sync_copy(acc_s.at[pl.ds(roff, rows_per)],
                        out_hbm.at[cid, pl.ds(roff, rows_per)])
    return _agg


_agg16 = _make_agg(F_H)
_agg8 = _make_agg(F_OUT_P)


def _matmul_body(xt_ref, w_ref, h_ref):
    h_ref[...] = lax.dot_general(
        xt_ref[...], w_ref[...], (((0,), (0,)), ((), ())),
        preferred_element_type=jnp.float32)


def _matmul(xt, W1):
    return pl.pallas_call(
        _matmul_body,
        grid=(GRID_N,),
        in_specs=[
            pl.BlockSpec((F_IN, ROW_T), lambda i: (0, i)),
            pl.BlockSpec((F_IN, F_H), lambda i: (0, 0)),
        ],
        out_specs=pl.BlockSpec((ROW_T, F_H), lambda i: (i, 0)),
        out_shape=jax.ShapeDtypeStruct((N, F_H), jnp.float32),
    )(xt, W1)


def _prep_body(h_ref, degp_ref, hp_ref):
    deg = jnp.sum(degp_ref[...], axis=0) + 1.0
    dinv = lax.rsqrt(deg)
    hp_ref[...] = h_ref[...] * dinv[:, None]


def _prep(h, deg_parts):
    return pl.pallas_call(
        _prep_body,
        grid=(GRID_B,),
        in_specs=[
            pl.BlockSpec((ROW_B, F_H), lambda i: (i, 0)),
            pl.BlockSpec((NW, ROW_B), lambda i: (0, i)),
        ],
        out_specs=pl.BlockSpec((ROW_B, F_H), lambda i: (i, 0)),
        out_shape=jax.ShapeDtypeStruct((N, F_H), jnp.float32),
    )(h, deg_parts)


def _mid_body(agg_ref, hp_ref, degp_ref, w2_ref, b1_ref, gp_ref):
    s = agg_ref[0] + agg_ref[1] + hp_ref[...]
    dinv = lax.rsqrt(jnp.sum(degp_ref[...], axis=0) + 1.0)[:, None]
    out1 = s * dinv + b1_ref[...]
    r = jnp.maximum(out1, 0.0)
    g = jnp.dot(r, w2_ref[...], preferred_element_type=jnp.float32)
    gp_ref[...] = g * dinv


def _mid(agg1, hp, deg_parts, W2p, b1r):
    return pl.pallas_call(
        _mid_body,
        grid=(GRID_B,),
        in_specs=[
            pl.BlockSpec((NC, ROW_B, F_H), lambda i: (0, i, 0)),
            pl.BlockSpec((ROW_B, F_H), lambda i: (i, 0)),
            pl.BlockSpec((NW, ROW_B), lambda i: (0, i)),
            pl.BlockSpec((F_H, F_OUT_P), lambda i: (0, 0)),
            pl.BlockSpec((1, F_H), lambda i: (0, 0)),
        ],
        out_specs=pl.BlockSpec((ROW_B, F_OUT_P), lambda i: (i, 0)),
        out_shape=jax.ShapeDtypeStruct((N, F_OUT_P), jnp.float32),
    )(agg1, hp, deg_parts, W2p, b1r)


def _final_body(agg_ref, gp_ref, degp_ref, b2_ref, out_ref):
    s = agg_ref[0] + agg_ref[1] + gp_ref[...]
    dinv = lax.rsqrt(jnp.sum(degp_ref[...], axis=0) + 1.0)[:, None]
    z = s * dinv + b2_ref[...]
    col = lax.broadcasted_iota(jnp.int32, (ROW_B, F_OUT_P), 1)
    zm = jnp.where(col < F_OUT, z, -1e30)
    m = jnp.max(zm, axis=1, keepdims=True)
    ez = jnp.exp(zm - m)
    ssum = jnp.sum(ez, axis=1, keepdims=True)
    ls = (z - m) - jnp.log(ssum)
    out_ref[...] = ls[:, :F_OUT].T


def _final(agg2, gp, deg_parts, b2r):
    return pl.pallas_call(
        _final_body,
        grid=(GRID_B,),
        in_specs=[
            pl.BlockSpec((NC, ROW_B, F_OUT_P), lambda i: (0, i, 0)),
            pl.BlockSpec((ROW_B, F_OUT_P), lambda i: (i, 0)),
            pl.BlockSpec((NW, ROW_B), lambda i: (0, i)),
            pl.BlockSpec((1, F_OUT_P), lambda i: (0, 0)),
        ],
        out_specs=pl.BlockSpec((F_OUT, ROW_B), lambda i: (0, i)),
        out_shape=jax.ShapeDtypeStruct((F_OUT, N), jnp.float32),
    )(agg2, gp, deg_parts, b2r)


def kernel(x, edge_index, W1, b1, W2, b2):
    edges3 = jnp.transpose(edge_index.reshape(2, NCHUNKS, CH), (1, 0, 2))

    deg_parts = _deg_kernel(edges3)
    h = _matmul(x.T, W1)
    hp = _prep(h, deg_parts)

    z16 = jnp.zeros((P, F_H), jnp.float32)
    agg1 = _agg16(edges3, hp, z16)

    W2p = jnp.zeros((F_H, F_OUT_P), jnp.float32).at[:, :F_OUT].set(W2)
    b1r = b1.reshape(1, F_H)
    gp = _mid(agg1, hp, deg_parts, W2p, b1r)

    z8 = jnp.zeros((P, F_OUT_P), jnp.float32)
    agg2 = _agg8(edges3, gp, z8)

    b2r = jnp.zeros((1, F_OUT_P), jnp.float32).at[0, :F_OUT].set(b2)
    return _final(agg2, gp, deg_parts, b2r).T
